# baseline (speedup 1.0000x reference)
import functools

import jax
import jax.numpy as jnp
from jax import lax
from jax.experimental import pallas as pl
from jax.experimental.pallas import tpu as pltpu
from jax.experimental.pallas import tpu_sc as plsc

import numpy as np

N = 10000
E = 160000
D = 128
T = 4
F_IN = 128
F_OUT = 32
EDGE_DIM = 16
AVG_DEG_LOG = float(np.log(17.0))


_SC_INFO = plsc.get_sparse_core_info()
_NC = _SC_INFO.num_cores
_NS = _SC_INFO.num_subcores
_NW = _NC * _NS
_EPW = E // _NW
_L = 16


_MESH = plsc.VectorSubcoreMesh(core_axis_name="c", subcore_axis_name="s")
_SC_PARAMS = pltpu.CompilerParams(needs_layout_passes=False)
_NVEC = 313
_EPW_PAD = _NVEC * _L
_EBUF = 6144
E_PAD = E + _EBUF
_NPT = 384
_NACC = _NW * _NPT
_NNS = _NACC + 112
_F = 512
_CH = 16


def _wid():
    return lax.axis_index("s") * _NC + lax.axis_index("c")


def _vec_loop(ref_stmts, n):
    lax.fori_loop(0, n, ref_stmts, None)


def _deg_hist(dst):
    @functools.partial(
        pl.kernel, mesh=_MESH,
        out_type=jax.ShapeDtypeStruct((_NW, _NACC), jnp.int32),
        compiler_params=_SC_PARAMS,
        scratch_types=[
            pltpu.VMEM((_EPW_PAD,), jnp.int32),
            pltpu.VMEM((_NACC,), jnp.int32),
            pltpu.SemaphoreType.DMA,
        ],
    )
    def k(dst_hbm, out_hbm, dstv, hist, sem):
        wid = _wid()
        base = wid * _EPW
        pltpu.sync_copy(dst_hbm.at[pl.ds(base, _EPW)], dstv.at[pl.ds(0, _EPW)])

        def zero_body(i, _):
            hist[pl.ds(i * _L, _L)] = jnp.zeros((_L,), jnp.int32)
        _vec_loop(zero_body, _NACC // _L)

        ones = jnp.full((_L,), 1, jnp.int32)
        iota = lax.iota(jnp.int32, _L)

        def body(i, _):
            idx = dstv[pl.ds(i * _L, _L)]
            plsc.addupdate_scatter(hist, [idx], ones)
        _vec_loop(body, _EPW // _L)
        tail = _EPW - (_EPW // _L) * _L
        idx = dstv[pl.ds((_EPW // _L) * _L, _L)]
        mask = iota < tail
        idx = jnp.where(mask, idx, 0)
        plsc.addupdate_scatter(hist, [idx], ones, mask=mask)
        pltpu.sync_copy(hist, out_hbm.at[wid])

    return k(dst)


def _cursor_build(hist):
    @functools.partial(
        pl.kernel, mesh=_MESH,
        out_type=(
            jax.ShapeDtypeStruct((_NW, _NACC), jnp.int32),
            jax.ShapeDtypeStruct((_NW, _L), jnp.int32),
            jax.ShapeDtypeStruct((_NACC,), jnp.float32),
        ),
        compiler_params=_SC_PARAMS,
        scratch_types=[
            pltpu.VMEM((_NW, _NPT), jnp.int32),
            pltpu.VMEM((_NW, _NPT), jnp.int32),
            pltpu.VMEM((_NPT,), jnp.int32),
            pltpu.VMEM((_NPT,), jnp.int32),
            pltpu.VMEM((_NPT,), jnp.float32),
            pltpu.VMEM((_L,), jnp.int32),
        ],
    )
    def k(hist_hbm, curs_hbm, st_hbm, deg_hbm, hloc, cbuf, tot, gs_span,
          degl, stv):
        wid = _wid()
        nlo = wid * _NPT
        nv = _NPT // _L
        pltpu.sync_copy(hist_hbm.at[:, pl.ds(nlo, _NPT)], hloc)

        def col(v, _):
            s = pl.ds(v * _L, _L)
            acc = jnp.zeros((_L,), jnp.int32)
            for w2 in range(_NW):
                cbuf[w2, s] = acc
                acc = acc + hloc[w2, s]
            tot[s] = acc
            degl[s] = acc.astype(jnp.float32)
        _vec_loop(col, nv)

        def cs(v, carry):
            s = pl.ds(v * _L, _L)
            t = tot[s]
            c = plsc.cumsum(t)
            gs_span[s] = c - t + carry
            return carry + jnp.sum(t)
        span_total = lax.fori_loop(0, nv, cs, jnp.int32(0))

        def fix(v, _):
            s = pl.ds(v * _L, _L)
            g = gs_span[s]
            for w2 in range(_NW):
                cbuf[w2, s] = cbuf[w2, s] + g
        _vec_loop(fix, nv)

        stv[...] = jnp.broadcast_to(span_total, (_L,))
        pltpu.sync_copy(cbuf, curs_hbm.at[:, pl.ds(nlo, _NPT)])
        pltpu.sync_copy(stv, st_hbm.at[wid])
        pltpu.sync_copy(degl, deg_hbm.at[pl.ds(nlo, _NPT)])

    return k(hist)


def _scatter_sort(dst, src, curs, st):
    @functools.partial(
        pl.kernel, mesh=_MESH,
        out_type=(
            jax.ShapeDtypeStruct((E_PAD,), jnp.int32),
            jax.ShapeDtypeStruct((E_PAD,), jnp.int32),
            jax.ShapeDtypeStruct((E_PAD,), jnp.int32),
            jax.ShapeDtypeStruct((_NNS,), jnp.int32),
        ),
        compiler_params=_SC_PARAMS,
        scratch_types=[
            pltpu.VMEM((_EPW_PAD,), jnp.int32),
            pltpu.VMEM((_EPW_PAD,), jnp.int32),
            pltpu.VMEM((_NACC,), jnp.int32),
            pltpu.VMEM((_NW, _L), jnp.int32),
            pltpu.VMEM((40, 128), jnp.int32),
            pltpu.VMEM((40 * 128,), jnp.int32),
            pltpu.VMEM((40 * 128,), jnp.int32),
            pltpu.VMEM((40 * 128,), jnp.int32),
            pltpu.VMEM((_L,), jnp.int32),
            pltpu.VMEM((112,), jnp.int32),
            pltpu.SemaphoreType.DMA,
        ],
    )
    def k(dst_hbm, src_hbm, curs_hbm, st_hbm, perm_hbm, psrc_hbm, pdst_hbm,
          ns_hbm, dstv, srcv, cursor, stbuf, stage_pos, stage_e, stage_s,
          stage_d, shuf, padv, sem):
        wid = _wid()
        base = wid * _EPW
        iota = lax.iota(jnp.int32, _L)
        pltpu.sync_copy(dst_hbm.at[pl.ds(base, _EPW)], dstv.at[pl.ds(0, _EPW)])
        pltpu.sync_copy(src_hbm.at[pl.ds(base, _EPW)], srcv.at[pl.ds(0, _EPW)])
        pltpu.sync_copy(curs_hbm.at[wid], cursor)
        pltpu.sync_copy(st_hbm, stbuf)

        sb = jnp.int32(0)
        for sp in range(_NW):
            def fixv(v, _, sp=sp, sbv=sb):
                s = pl.ds(sp * _NPT + v * _L, _L)
                cursor[s] = cursor[s] + sbv
            _vec_loop(fixv, _NPT // _L)
            sb = sb + stbuf[sp][0]

        @pl.when(wid == 0)
        def _emit():
            def pv(i, _):
                padv[pl.ds(i * _L, _L)] = jnp.full((_L,), E, jnp.int32)
            _vec_loop(pv, 112 // _L)
            pltpu.sync_copy(cursor, ns_hbm.at[pl.ds(0, _NACC)])
            pltpu.sync_copy(padv, ns_hbm.at[pl.ds(_NACC, 112)])

        def padinit(i, _):
            r = i // 8
            cv8 = i % 8
            stage_pos[r, pl.ds(cv8 * _L, _L)] = E + iota
        _vec_loop(padinit, 40 * 8)

        ones = jnp.full((_L,), 1, jnp.int32)

        def sbody(i, _):
            valid = (i * _L + iota) < _EPW
            d_raw = dstv[pl.ds(i * _L, _L)]
            s_raw = srcv[pl.ds(i * _L, _L)]
            key = jnp.where(valid, d_raw, jnp.int32(0x7FFFFFFF))
            dsort, ls = plsc.sort_key_val(key, iota)
            shuf[...] = dsort
            prev = plsc.load_gather(shuf, [jnp.maximum(iota - 1, 0)])
            runstart = (iota == 0) | (dsort != prev)
            rsidx = jnp.where(runstart, iota, 0)
            rank = iota - plsc.cummax(rsidx)
            dsafe = jnp.minimum(dsort, N - 1)
            c0 = plsc.load_gather(cursor, [dsafe])
            okv = dsort < N
            pos = jnp.where(okv, c0 + rank, E + iota)
            e_sorted = base + i * _L + ls
            shuf[...] = s_raw
            s_sorted = plsc.load_gather(shuf, [ls])
            stage_pos[i // 8, pl.ds((i % 8) * _L, _L)] = pos
            stage_e[pl.ds(i * _L, _L)] = e_sorted
            stage_s[pl.ds(i * _L, _L)] = s_sorted
            stage_d[pl.ds(i * _L, _L)] = dsafe
            plsc.addupdate_scatter(cursor, [jnp.where(valid, d_raw, 0)], ones,
                                   mask=valid)
        _vec_loop(sbody, _NVEC)

        copies = []
        for c in range(40):
            sl = pl.ds(c * 128, 128)
            idx = stage_pos.at[c]
            copies.append(pltpu.async_copy(stage_e.at[sl], perm_hbm.at[idx], sem))
            copies.append(pltpu.async_copy(stage_s.at[sl], psrc_hbm.at[idx], sem))
            copies.append(pltpu.async_copy(stage_d.at[sl], pdst_hbm.at[idx], sem))
        for cp in copies:
            cp.wait()

    return k(dst, src, curs, st)


def _accumulate(Bm, Cm, perm, psrc, pdst, nodestart):
    @functools.partial(
        pl.kernel, mesh=_MESH,
        out_type=tuple(jax.ShapeDtypeStruct((_NACC, _F), jnp.float32)
                       for _ in range(4)),
        compiler_params=_SC_PARAMS,
        scratch_types=[
            pltpu.VMEM((400,), jnp.int32),
            pltpu.VMEM((_EBUF,), jnp.int32),
            pltpu.VMEM((_EBUF,), jnp.int32),
            pltpu.VMEM((_EBUF,), jnp.int32),
            pltpu.VMEM((2, _CH, _F), jnp.float32),
            pltpu.VMEM((2, _CH, _F), jnp.float32),
            pltpu.VMEM((32, _F), jnp.float32),
            pltpu.VMEM((32, _F), jnp.float32),
            pltpu.VMEM((32, _F), jnp.float32),
            pltpu.VMEM((32, _F), jnp.float32),
            pltpu.SemaphoreType.DMA((2,)),
        ],
    )
    def k(b_hbm, c_hbm, perm_hbm, psrc_hbm, pdst_hbm, ns_hbm,
          s1_hbm, s2_hbm, mx_hbm, mn_hbm,
          nsv, psrcv, permv, pdstv, bbuf, cbuf, s1a, s2a, mxa, mna, semg):
        wid = _wid()
        nlo = wid * _NPT
        iota = lax.iota(jnp.int32, _L)
        pltpu.sync_copy(ns_hbm.at[pl.ds(nlo, 392)], nsv.at[pl.ds(0, 392)])
        estart = nsv[pl.ds(0, _L)][0]
        eend = nsv[pl.ds(_NPT, _L)][0]
        a0 = (estart // 8) * 8
        d0 = estart - a0
        cnt = eend - a0
        pltpu.sync_copy(psrc_hbm.at[pl.ds(a0, _EBUF)], psrcv)
        pltpu.sync_copy(perm_hbm.at[pl.ds(a0, _EBUF)], permv)
        pltpu.sync_copy(pdst_hbm.at[pl.ds(a0, _EBUF)], pdstv)

        def sanitize(i, _):
            s = pl.ds(i * _L, _L)
            ok = (i * _L + iota) < cnt
            psrcv[s] = jnp.where(ok, psrcv[s], 0)
            permv[s] = jnp.where(ok, permv[s], 0)
        _vec_loop(sanitize, _EBUF // _L)

        def init_acc():
            def ib(i, _):
                n = i // 32
                s = pl.ds((i % 32) * _L, _L)
                s1a[n, s] = jnp.zeros((_L,), jnp.float32)
                s2a[n, s] = jnp.zeros((_L,), jnp.float32)
                mxa[n, s] = jnp.full((_L,), -3.0e38, jnp.float32)
                mna[n, s] = jnp.full((_L,), 3.0e38, jnp.float32)
            _vec_loop(ib, 32 * (_F // _L))

        init_acc()

        nch = (eend - a0 + _CH - 1) // _CH

        def fire(ci):
            par = lax.rem(ci, 2)
            off = ci * _CH
            pltpu.async_copy(b_hbm.at[psrcv.at[pl.ds(off, _CH)]],
                             bbuf.at[par], semg.at[par])
            pltpu.async_copy(c_hbm.at[permv.at[pl.ds(off, _CH)]],
                             cbuf.at[par], semg.at[par])

        @pl.when(nch > 0)
        def _p0():
            fire(0)

        def flush(cur):
            row0 = nlo + cur * 32
            pltpu.sync_copy(s1a, s1_hbm.at[pl.ds(row0, 32)])
            pltpu.sync_copy(s2a, s2_hbm.at[pl.ds(row0, 32)])
            pltpu.sync_copy(mxa, mx_hbm.at[pl.ds(row0, 32)])
            pltpu.sync_copy(mna, mn_hbm.at[pl.ds(row0, 32)])

        def chunk_body(ci, cur):
            par = lax.rem(ci, 2)

            @pl.when(ci + 1 < nch)
            def _pref():
                fire(ci + 1)

            pltpu.make_async_copy(b_hbm.at[pl.ds(0, _CH)], bbuf.at[par],
                                  semg.at[par]).wait()
            pltpu.make_async_copy(c_hbm.at[pl.ds(0, _CH)], cbuf.at[par],
                                  semg.at[par]).wait()

            gbase = a0 + ci * _CH
            nodes16 = pdstv[pl.ds(ci * _CH, _L)]
            for e in range(_CH):
                p = gbase + e
                valid = (p >= estart) & (p < eend)
                node = nodes16[e]
                grp = lax.div(node - nlo, jnp.int32(32))
                do_flush = valid & (grp != cur)

                @pl.when(do_flush)
                def _fl(cur=cur):
                    flush(cur)
                    init_acc()

                cur = jnp.where(valid, grp, cur)
                locs = jnp.where(valid, node - nlo - grp * 32, 0)

                @pl.when(valid)
                def _acc(par=par, e=e, locs=locs):
                    def sub(sb, _):
                        s = pl.ds(sb * _L, _L)
                        mv = bbuf[par, e, s] + cbuf[par, e, s]
                        plsc.addupdate(s1a.at[locs, s], mv)
                        plsc.addupdate(s2a.at[locs, s], mv * mv)
                        mxa[locs, s] = jnp.maximum(mxa[locs, s], mv)
                        mna[locs, s] = jnp.minimum(mna[locs, s], mv)
                    lax.fori_loop(0, _F // _L, sub, None)
            return cur

        cur_fin = lax.fori_loop(0, nch, chunk_body, jnp.int32(0))
        flush(cur_fin)

    return k(Bm, Cm, perm, psrc, pdst, nodestart)


def _post_body(o_ref, wl_ref, bl_ref, g_ref, be_ref, out_ref):
    y = jnp.dot(o_ref[...], wl_ref[...], preferred_element_type=jnp.float32)
    y = y + bl_ref[...]
    mu = jnp.mean(y, axis=0, keepdims=True)
    v = jnp.mean(y * y, axis=0, keepdims=True) - mu * mu
    yn = (y - mu) * lax.rsqrt(v + 1e-5) * g_ref[...] + be_ref[...]
    out_ref[...] = jnp.maximum(yn, 0.0)


def kernel(x, edge_attr, W_e, b_e, W_pre, b_pre, W_post, b_post, W_lin, b_lin, gamma, beta, edge_index):
    src = edge_index[0].astype(jnp.int32)
    dst = edge_index[1].astype(jnp.int32)

    Wd = jnp.concatenate([W_pre[t, 0:F_IN] for t in range(T)], axis=1)
    Ws = jnp.concatenate([W_pre[t, F_IN:2 * F_IN] for t in range(T)], axis=1)
    We2 = jnp.concatenate([W_pre[t, 2 * F_IN:3 * F_IN] for t in range(T)], axis=1)
    M = W_e @ We2
    cbias = b_e @ We2 + jnp.concatenate([b_pre[t] for t in range(T)])

    A = x @ Wd
    B = x @ Ws
    C = edge_attr @ M + cbias

    hist = _deg_hist(dst)
    curs, st, deg = _cursor_build(hist)
    perm, psrc, pdst, nodestart = _scatter_sort(dst, src, curs, st)
    S1, S2, MX, MN = _accumulate(B, C, perm, psrc, pdst, nodestart)
    S1, S2, MX, MN = S1[:N], S2[:N], MX[:N], MN[:N]
    deg = deg[:N]
    deg_c = jnp.maximum(deg, 1.0)[:, None]
    log_deg = jnp.log(jnp.maximum(deg, 1.0) + 1.0)[:, None]
    amp = log_deg / AVG_DEG_LOG
    att = AVG_DEG_LOG / log_deg
    has = (deg > 0)[:, None]

    mean = jnp.where(has, (deg[:, None] * A + S1) / deg_c, 0.0)
    mx = jnp.where(has, A + MX, 0.0)
    mn = jnp.where(has, A + MN, 0.0)
    var = jnp.where(has, S2 / deg_c - (S1 / deg_c) ** 2, 0.0)
    std = jnp.sqrt(jax.nn.relu(var) + 1e-5)

    touts = []
    for t in range(T):
        sl = slice(t * F_IN, (t + 1) * F_IN)
        agg = jnp.concatenate([mean[:, sl], mx[:, sl], mn[:, sl], std[:, sl]], axis=-1)
        scaled = jnp.concatenate([agg, agg * amp, agg * att], axis=-1)
        tower_in = jnp.concatenate([x, scaled], axis=-1)
        touts.append(tower_in @ W_post[t] + b_post[t])
    o = jnp.concatenate(touts, axis=-1)

    out = pl.pallas_call(
        _post_body,
        out_shape=jax.ShapeDtypeStruct((N, T * F_OUT), jnp.float32),
    )(o, W_lin, b_lin, gamma, beta)
    return out


# speedup vs baseline: 1.5605x; 1.5605x over previous
import functools

import jax
import jax.numpy as jnp
from jax import lax
from jax.experimental import pallas as pl
from jax.experimental.pallas import tpu as pltpu
from jax.experimental.pallas import tpu_sc as plsc

import numpy as np

N = 10000
E = 160000
D = 128
T = 4
F_IN = 128
F_OUT = 32
EDGE_DIM = 16
AVG_DEG_LOG = float(np.log(17.0))


_SC_INFO = plsc.get_sparse_core_info()
_NC = _SC_INFO.num_cores
_NS = _SC_INFO.num_subcores
_NW = _NC * _NS
_EPW = E // _NW
_L = 16


_MESH = plsc.VectorSubcoreMesh(core_axis_name="c", subcore_axis_name="s")
_SC_PARAMS = pltpu.CompilerParams(needs_layout_passes=False)
_NVEC = 313
_EPW_PAD = _NVEC * _L
_EBUF = 6144
E_PAD = E + _EBUF
_NPT = 384
_NPT3 = 320
_NACC = _NW * _NPT
_NNS = _NACC + 112
_F = 512
_CH = 16


def _wid():
    return lax.axis_index("s") * _NC + lax.axis_index("c")


def _vec_loop(ref_stmts, n):
    lax.fori_loop(0, n, ref_stmts, None)


def _deg_hist(dst):
    @functools.partial(
        pl.kernel, mesh=_MESH,
        out_type=jax.ShapeDtypeStruct((_NW, _NACC), jnp.int32),
        compiler_params=_SC_PARAMS,
        scratch_types=[
            pltpu.VMEM((_EPW_PAD,), jnp.int32),
            pltpu.VMEM((_NACC,), jnp.int32),
            pltpu.SemaphoreType.DMA,
        ],
    )
    def k(dst_hbm, out_hbm, dstv, hist, sem):
        wid = _wid()
        base = wid * _EPW
        pltpu.sync_copy(dst_hbm.at[pl.ds(base, _EPW)], dstv.at[pl.ds(0, _EPW)])

        def zero_body(i, _):
            hist[pl.ds(i * _L, _L)] = jnp.zeros((_L,), jnp.int32)
        _vec_loop(zero_body, _NACC // _L)

        ones = jnp.full((_L,), 1, jnp.int32)
        iota = lax.iota(jnp.int32, _L)

        def body(i, _):
            idx = dstv[pl.ds(i * _L, _L)]
            plsc.addupdate_scatter(hist, [idx], ones)
        _vec_loop(body, _EPW // _L)
        tail = _EPW - (_EPW // _L) * _L
        idx = dstv[pl.ds((_EPW // _L) * _L, _L)]
        mask = iota < tail
        idx = jnp.where(mask, idx, 0)
        plsc.addupdate_scatter(hist, [idx], ones, mask=mask)
        pltpu.sync_copy(hist, out_hbm.at[wid])

    return k(dst)


def _cursor_build(hist):
    @functools.partial(
        pl.kernel, mesh=_MESH,
        out_type=(
            jax.ShapeDtypeStruct((_NW, _NACC), jnp.int32),
            jax.ShapeDtypeStruct((_NW, _L), jnp.int32),
            jax.ShapeDtypeStruct((_NACC,), jnp.float32),
        ),
        compiler_params=_SC_PARAMS,
        scratch_types=[
            pltpu.VMEM((_NW, _NPT), jnp.int32),
            pltpu.VMEM((_NW, _NPT), jnp.int32),
            pltpu.VMEM((_NPT,), jnp.int32),
            pltpu.VMEM((_NPT,), jnp.int32),
            pltpu.VMEM((_NPT,), jnp.float32),
            pltpu.VMEM((_L,), jnp.int32),
        ],
    )
    def k(hist_hbm, curs_hbm, st_hbm, deg_hbm, hloc, cbuf, tot, gs_span,
          degl, stv):
        wid = _wid()
        nlo = wid * _NPT
        nv = _NPT // _L
        pltpu.sync_copy(hist_hbm.at[:, pl.ds(nlo, _NPT)], hloc)

        def col(v, _):
            s = pl.ds(v * _L, _L)
            acc = jnp.zeros((_L,), jnp.int32)
            for w2 in range(_NW):
                cbuf[w2, s] = acc
                acc = acc + hloc[w2, s]
            tot[s] = acc
            degl[s] = acc.astype(jnp.float32)
        _vec_loop(col, nv)

        def cs(v, carry):
            s = pl.ds(v * _L, _L)
            t = tot[s]
            c = plsc.cumsum(t)
            gs_span[s] = c - t + carry
            return carry + jnp.sum(t)
        span_total = lax.fori_loop(0, nv, cs, jnp.int32(0))

        def fix(v, _):
            s = pl.ds(v * _L, _L)
            g = gs_span[s]
            for w2 in range(_NW):
                cbuf[w2, s] = cbuf[w2, s] + g
        _vec_loop(fix, nv)

        stv[...] = jnp.broadcast_to(span_total, (_L,))
        pltpu.sync_copy(cbuf, curs_hbm.at[:, pl.ds(nlo, _NPT)])
        pltpu.sync_copy(stv, st_hbm.at[wid])
        pltpu.sync_copy(degl, deg_hbm.at[pl.ds(nlo, _NPT)])

    return k(hist)


def _scatter_sort(dst, src, curs, st):
    @functools.partial(
        pl.kernel, mesh=_MESH,
        out_type=(
            jax.ShapeDtypeStruct((E_PAD,), jnp.int32),
            jax.ShapeDtypeStruct((E_PAD,), jnp.int32),
            jax.ShapeDtypeStruct((E_PAD,), jnp.int32),
            jax.ShapeDtypeStruct((_NNS,), jnp.int32),
        ),
        compiler_params=_SC_PARAMS,
        scratch_types=[
            pltpu.VMEM((_EPW_PAD,), jnp.int32),
            pltpu.VMEM((_EPW_PAD,), jnp.int32),
            pltpu.VMEM((_NACC,), jnp.int32),
            pltpu.VMEM((_NW, _L), jnp.int32),
            pltpu.VMEM((40, 128), jnp.int32),
            pltpu.VMEM((40 * 128,), jnp.int32),
            pltpu.VMEM((40 * 128,), jnp.int32),
            pltpu.VMEM((40 * 128,), jnp.int32),
            pltpu.VMEM((_L,), jnp.int32),
            pltpu.VMEM((112,), jnp.int32),
            pltpu.SemaphoreType.DMA,
        ],
    )
    def k(dst_hbm, src_hbm, curs_hbm, st_hbm, perm_hbm, psrc_hbm, pdst_hbm,
          ns_hbm, dstv, srcv, cursor, stbuf, stage_pos, stage_e, stage_s,
          stage_d, shuf, padv, sem):
        wid = _wid()
        base = wid * _EPW
        iota = lax.iota(jnp.int32, _L)
        pltpu.sync_copy(dst_hbm.at[pl.ds(base, _EPW)], dstv.at[pl.ds(0, _EPW)])
        pltpu.sync_copy(src_hbm.at[pl.ds(base, _EPW)], srcv.at[pl.ds(0, _EPW)])
        pltpu.sync_copy(curs_hbm.at[wid], cursor)
        pltpu.sync_copy(st_hbm, stbuf)

        sb = jnp.int32(0)
        for sp in range(_NW):
            def fixv(v, _, sp=sp, sbv=sb):
                s = pl.ds(sp * _NPT + v * _L, _L)
                cursor[s] = cursor[s] + sbv
            _vec_loop(fixv, _NPT // _L)
            sb = sb + stbuf[sp][0]

        @pl.when(wid == 0)
        def _emit():
            def pv(i, _):
                padv[pl.ds(i * _L, _L)] = jnp.full((_L,), E, jnp.int32)
            _vec_loop(pv, 112 // _L)
            pltpu.sync_copy(cursor, ns_hbm.at[pl.ds(0, _NACC)])
            pltpu.sync_copy(padv, ns_hbm.at[pl.ds(_NACC, 112)])

        def padinit(i, _):
            r = i // 8
            cv8 = i % 8
            stage_pos[r, pl.ds(cv8 * _L, _L)] = E + iota
        _vec_loop(padinit, 40 * 8)

        ones = jnp.full((_L,), 1, jnp.int32)

        def sbody(i, _):
            valid = (i * _L + iota) < _EPW
            d_raw = dstv[pl.ds(i * _L, _L)]
            s_raw = srcv[pl.ds(i * _L, _L)]
            key = jnp.where(valid, d_raw, jnp.int32(0x7FFFFFFF))
            dsort, ls = plsc.sort_key_val(key, iota)
            shuf[...] = dsort
            prev = plsc.load_gather(shuf, [jnp.maximum(iota - 1, 0)])
            runstart = (iota == 0) | (dsort != prev)
            rsidx = jnp.where(runstart, iota, 0)
            rank = iota - plsc.cummax(rsidx)
            dsafe = jnp.minimum(dsort, N - 1)
            c0 = plsc.load_gather(cursor, [dsafe])
            okv = dsort < N
            pos = jnp.where(okv, c0 + rank, E + iota)
            e_sorted = base + i * _L + ls
            shuf[...] = s_raw
            s_sorted = plsc.load_gather(shuf, [ls])
            stage_pos[i // 8, pl.ds((i % 8) * _L, _L)] = pos
            stage_e[pl.ds(i * _L, _L)] = e_sorted
            stage_s[pl.ds(i * _L, _L)] = s_sorted
            stage_d[pl.ds(i * _L, _L)] = dsafe
            plsc.addupdate_scatter(cursor, [jnp.where(valid, d_raw, 0)], ones,
                                   mask=valid)
        _vec_loop(sbody, _NVEC)

        copies = []
        for c in range(40):
            sl = pl.ds(c * 128, 128)
            idx = stage_pos.at[c]
            copies.append(pltpu.async_copy(stage_e.at[sl], perm_hbm.at[idx], sem))
            copies.append(pltpu.async_copy(stage_s.at[sl], psrc_hbm.at[idx], sem))
            copies.append(pltpu.async_copy(stage_d.at[sl], pdst_hbm.at[idx], sem))
        for cp in copies:
            cp.wait()

    return k(dst, src, curs, st)


def _accumulate(Bm, Cm, perm, psrc, pdst, nodestart):
    @functools.partial(
        pl.kernel, mesh=_MESH,
        out_type=tuple(jax.ShapeDtypeStruct((_NACC, _F), jnp.float32)
                       for _ in range(4)),
        compiler_params=_SC_PARAMS,
        scratch_types=[
            pltpu.VMEM((400,), jnp.int32),
            pltpu.VMEM((_EBUF,), jnp.int32),
            pltpu.VMEM((_EBUF,), jnp.int32),
            pltpu.VMEM((_EBUF,), jnp.int32),
            pltpu.VMEM((2, _CH, _F), jnp.float32),
            pltpu.VMEM((2, _CH, _F), jnp.float32),
            pltpu.VMEM((32, _F), jnp.float32),
            pltpu.VMEM((32, _F), jnp.float32),
            pltpu.VMEM((32, _F), jnp.float32),
            pltpu.VMEM((32, _F), jnp.float32),
            pltpu.SemaphoreType.DMA((2,)),
        ],
    )
    def k(b_hbm, c_hbm, perm_hbm, psrc_hbm, pdst_hbm, ns_hbm,
          s1_hbm, s2_hbm, mx_hbm, mn_hbm,
          nsv, psrcv, permv, pdstv, bbuf, cbuf, s1a, s2a, mxa, mna, semg):
        wid = _wid()
        nlo = wid * _NPT3
        iota = lax.iota(jnp.int32, _L)
        pltpu.sync_copy(ns_hbm.at[pl.ds(nlo, 328)], nsv.at[pl.ds(0, 328)])
        estart = nsv[pl.ds(0, _L)][0]
        eend = nsv[pl.ds(_NPT3, _L)][0]
        a0 = (estart // 8) * 8
        d0 = estart - a0
        cnt = eend - a0
        pltpu.sync_copy(psrc_hbm.at[pl.ds(a0, _EBUF)], psrcv)
        pltpu.sync_copy(perm_hbm.at[pl.ds(a0, _EBUF)], permv)
        pltpu.sync_copy(pdst_hbm.at[pl.ds(a0, _EBUF)], pdstv)

        def sanitize(i, _):
            s = pl.ds(i * _L, _L)
            ok = (i * _L + iota) < cnt
            psrcv[s] = jnp.where(ok, psrcv[s], 0)
            permv[s] = jnp.where(ok, permv[s], 0)
        _vec_loop(sanitize, _EBUF // _L)

        def init_acc():
            def ib(i, _):
                n = i // 32
                s = pl.ds((i % 32) * _L, _L)
                s1a[n, s] = jnp.zeros((_L,), jnp.float32)
                s2a[n, s] = jnp.zeros((_L,), jnp.float32)
                mxa[n, s] = jnp.full((_L,), -3.0e38, jnp.float32)
                mna[n, s] = jnp.full((_L,), 3.0e38, jnp.float32)
            _vec_loop(ib, 32 * (_F // _L))

        init_acc()

        nch = (eend - a0 + _CH - 1) // _CH

        def fire(ci):
            par = lax.rem(ci, 2)
            off = ci * _CH
            pltpu.async_copy(b_hbm.at[psrcv.at[pl.ds(off, _CH)]],
                             bbuf.at[par], semg.at[par])
            pltpu.async_copy(c_hbm.at[permv.at[pl.ds(off, _CH)]],
                             cbuf.at[par], semg.at[par])

        @pl.when(nch > 0)
        def _p0():
            fire(0)

        def flush(cur):
            row0 = nlo + cur * 32
            pltpu.sync_copy(s1a, s1_hbm.at[pl.ds(row0, 32)])
            pltpu.sync_copy(s2a, s2_hbm.at[pl.ds(row0, 32)])
            pltpu.sync_copy(mxa, mx_hbm.at[pl.ds(row0, 32)])
            pltpu.sync_copy(mna, mn_hbm.at[pl.ds(row0, 32)])

        def chunk_body(ci, cur):
            par = lax.rem(ci, 2)

            @pl.when(ci + 1 < nch)
            def _pref():
                fire(ci + 1)

            pltpu.make_async_copy(b_hbm.at[pl.ds(0, _CH)], bbuf.at[par],
                                  semg.at[par]).wait()
            pltpu.make_async_copy(c_hbm.at[pl.ds(0, _CH)], cbuf.at[par],
                                  semg.at[par]).wait()

            gbase = a0 + ci * _CH
            nodes16 = pdstv[pl.ds(ci * _CH, _L)]
            for e in range(_CH):
                p = gbase + e
                valid = (p >= estart) & (p < eend)
                node = nodes16[e]
                grp = lax.div(node - nlo, jnp.int32(32))
                do_flush = valid & (grp != cur)

                @pl.when(do_flush)
                def _fl(cur=cur):
                    flush(cur)
                    init_acc()

                cur = jnp.where(valid, grp, cur)
                locs = jnp.where(valid, node - nlo - grp * 32, 0)

                @pl.when(valid)
                def _acc(par=par, e=e, locs=locs):
                    def sub(sb, _):
                        s = pl.ds(sb * _L, _L)
                        mv = bbuf[par, e, s] + cbuf[par, e, s]
                        plsc.addupdate(s1a.at[locs, s], mv)
                        plsc.addupdate(s2a.at[locs, s], mv * mv)
                        mxa[locs, s] = jnp.maximum(mxa[locs, s], mv)
                        mna[locs, s] = jnp.minimum(mna[locs, s], mv)
                    lax.fori_loop(0, _F // _L, sub, None)
            return cur

        cur_fin = lax.fori_loop(0, nch, chunk_body, jnp.int32(0))
        flush(cur_fin)

    return k(Bm, Cm, perm, psrc, pdst, nodestart)


def _ab_body(x_ref, w_ref, o_ref):
    o_ref[...] = jnp.dot(x_ref[...], w_ref[...],
                         preferred_element_type=jnp.float32)


def _xw(x, Wab):
    nb = 10
    return pl.pallas_call(
        _ab_body,
        grid=(nb,),
        in_specs=[
            pl.BlockSpec((N // nb, D), lambda i: (i, 0)),
            pl.BlockSpec(Wab.shape, lambda i: (0, 0)),
        ],
        out_specs=pl.BlockSpec((N // nb, Wab.shape[1]), lambda i: (i, 0)),
        out_shape=jax.ShapeDtypeStruct((N, Wab.shape[1]), jnp.float32),
    )(x, Wab)


def _c_body(ea_ref, we_ref, we2_ref, be_ref, bp_ref, o_ref):
    M = jnp.dot(we_ref[...], we2_ref[...], preferred_element_type=jnp.float32)
    cb = jnp.dot(be_ref[...], we2_ref[...], preferred_element_type=jnp.float32)
    o_ref[...] = (jnp.dot(ea_ref[...], M, preferred_element_type=jnp.float32)
                  + cb + bp_ref[...])


def _cmat(edge_attr, W_e, We2, b_e2, bpre2):
    nb = 20
    return pl.pallas_call(
        _c_body,
        grid=(nb,),
        in_specs=[
            pl.BlockSpec((E // nb, EDGE_DIM), lambda i: (i, 0)),
            pl.BlockSpec((EDGE_DIM, D), lambda i: (0, 0)),
            pl.BlockSpec((D, _F), lambda i: (0, 0)),
            pl.BlockSpec((1, D), lambda i: (0, 0)),
            pl.BlockSpec((1, _F), lambda i: (0, 0)),
        ],
        out_specs=pl.BlockSpec((E // nb, _F), lambda i: (i, 0)),
        out_shape=jax.ShapeDtypeStruct((E, _F), jnp.float32),
    )(edge_attr, W_e, We2, b_e2, bpre2)


def _z_body(s1_ref, s2_ref, mx_ref, mn_ref, a_ref, x_ref, deg_ref,
            wx_ref, w1_ref, w2_ref, w3_ref, bp_ref, o_ref):
    deg = deg_ref[...]
    degc = jnp.maximum(deg, 1.0)
    has = deg > 0
    logd = jnp.log(degc + 1.0)
    amp = logd / AVG_DEG_LOG
    att = AVG_DEG_LOG / logd
    S1, S2 = s1_ref[...], s2_ref[...]
    A = a_ref[...]
    mean = jnp.where(has, (deg * A + S1) / degc, 0.0)
    mx = jnp.where(has, A + mx_ref[...], 0.0)
    mn = jnp.where(has, A + mn_ref[...], 0.0)
    var = jnp.where(has, S2 / degc - (S1 / degc) ** 2, 0.0)
    std = jnp.sqrt(jnp.maximum(var, 0.0) + 1e-5)
    xw = jnp.dot(x_ref[...], wx_ref[...], preferred_element_type=jnp.float32)
    outs = []
    for t in range(T):
        sl = slice(t * F_IN, (t + 1) * F_IN)
        agg = jnp.concatenate([mean[:, sl], mx[:, sl], mn[:, sl], std[:, sl]],
                              axis=-1)
        u1 = jnp.dot(agg, w1_ref[t], preferred_element_type=jnp.float32)
        u2 = jnp.dot(agg, w2_ref[t], preferred_element_type=jnp.float32)
        u3 = jnp.dot(agg, w3_ref[t], preferred_element_type=jnp.float32)
        outs.append(xw[:, t * F_OUT:(t + 1) * F_OUT] + u1 + amp * u2 + att * u3)
    o_ref[...] = jnp.concatenate(outs, axis=-1) + bp_ref[...]


def _zmat(S1, S2, MX, MN, A, x, deg2, Wx, W1, W2, W3, bpost2):
    nb = 10
    bm = N // nb
    bs = lambda shp: pl.BlockSpec(shp, lambda i: tuple(0 for _ in shp))
    return pl.pallas_call(
        _z_body,
        grid=(nb,),
        in_specs=[
            pl.BlockSpec((bm, _F), lambda i: (i, 0)),
            pl.BlockSpec((bm, _F), lambda i: (i, 0)),
            pl.BlockSpec((bm, _F), lambda i: (i, 0)),
            pl.BlockSpec((bm, _F), lambda i: (i, 0)),
            pl.BlockSpec((bm, _F), lambda i: (i, 0)),
            pl.BlockSpec((bm, D), lambda i: (i, 0)),
            pl.BlockSpec((bm, 1), lambda i: (i, 0)),
            bs((D, T * F_OUT)),
            bs((T, 4 * F_IN, F_OUT)),
            bs((T, 4 * F_IN, F_OUT)),
            bs((T, 4 * F_IN, F_OUT)),
            bs((1, T * F_OUT)),
        ],
        out_specs=pl.BlockSpec((bm, T * F_OUT), lambda i: (i, 0)),
        out_shape=jax.ShapeDtypeStruct((N, T * F_OUT), jnp.float32),
    )(S1, S2, MX, MN, A, x, deg2, Wx, W1, W2, W3, bpost2)


def _post_body(o_ref, wl_ref, bl_ref, g_ref, be_ref, out_ref):
    y = jnp.dot(o_ref[...], wl_ref[...], preferred_element_type=jnp.float32)
    y = y + bl_ref[...]
    mu = jnp.mean(y, axis=0, keepdims=True)
    v = jnp.mean(y * y, axis=0, keepdims=True) - mu * mu
    yn = (y - mu) * lax.rsqrt(v + 1e-5) * g_ref[...] + be_ref[...]
    out_ref[...] = jnp.maximum(yn, 0.0)


def kernel(x, edge_attr, W_e, b_e, W_pre, b_pre, W_post, b_post, W_lin, b_lin, gamma, beta, edge_index):
    src = edge_index[0].astype(jnp.int32)
    dst = edge_index[1].astype(jnp.int32)

    Wd = jnp.concatenate([W_pre[t, 0:F_IN] for t in range(T)], axis=1)
    Ws = jnp.concatenate([W_pre[t, F_IN:2 * F_IN] for t in range(T)], axis=1)
    We2 = jnp.concatenate([W_pre[t, 2 * F_IN:3 * F_IN] for t in range(T)], axis=1)
    Wab = jnp.concatenate([Wd, Ws], axis=1)
    b_e2 = b_e[None, :]
    bpre2 = jnp.concatenate([b_pre[t] for t in range(T)])[None, :]

    AB = _xw(x, Wab)
    A, B = AB[:, :_F], AB[:, _F:]
    C = _cmat(edge_attr, W_e, We2, b_e2, bpre2)

    hist = _deg_hist(dst)
    curs, st, deg = _cursor_build(hist)
    perm, psrc, pdst, nodestart = _scatter_sort(dst, src, curs, st)
    S1, S2, MX, MN = _accumulate(B, C, perm, psrc, pdst, nodestart)
    S1, S2, MX, MN = S1[:N], S2[:N], MX[:N], MN[:N]
    deg2 = deg[:N, None]

    Wx = jnp.concatenate([W_post[t, 0:F_IN] for t in range(T)], axis=1)
    W1 = W_post[:, F_IN:5 * F_IN, :]
    W2 = W_post[:, 5 * F_IN:9 * F_IN, :]
    W3 = W_post[:, 9 * F_IN:13 * F_IN, :]
    bpost2 = b_post.reshape(1, T * F_OUT)

    z = _zmat(S1, S2, MX, MN, A, x, deg2, Wx, W1, W2, W3, bpost2)

    out = pl.pallas_call(
        _post_body,
        out_shape=jax.ShapeDtypeStruct((N, T * F_OUT), jnp.float32),
    )(z, W_lin, b_lin, gamma, beta)
    return out
